# Initial kernel scaffold; baseline (speedup 1.0000x reference)
import jax
import jax.numpy as jnp
from jax.experimental import pallas as pl


def kernel(x, edge_index, W_emb, W0, b0, W1, b1, W2, b2):
    raise NotImplementedError("write your pallas kernel here")


# speedup vs baseline: 7.5569x; 7.5569x over previous
import functools

import jax
import jax.numpy as jnp
from jax import lax
from jax.experimental import pallas as pl
from jax.experimental.pallas import tpu as pltpu
from jax.experimental.pallas import tpu_sc as plsc

N = 10000
E = 320000
INP = 128
EMB = 64

NC = 2
NS = 16
NW = NC * NS
EPW = E // NW
CHUNK = 80
NCHUNK = EPW // CHUNK
RPT = 624
TAIL = N - RPT * NS

@functools.cache
def _make_spmm_sc():
    mesh = plsc.VectorSubcoreMesh(
        core_axis_name="c", subcore_axis_name="s",
        num_cores=NC, num_subcores=NS)

    @functools.partial(
        pl.kernel,
        out_type=jax.ShapeDtypeStruct((NC, N, EMB), jnp.float32),
        mesh=mesh,
        scratch_types=[
            pltpu.MemorySpace.VMEM_SHARED((N, EMB), jnp.float32),
            pltpu.VMEM((NCHUNK, CHUNK), jnp.int32),
            pltpu.VMEM((NCHUNK, CHUNK), jnp.int32),
            pltpu.VMEM((CHUNK, EMB), jnp.float32),
            pltpu.SemaphoreType.DMA,
        ],
        compiler_params=pltpu.CompilerParams(use_tc_tiling_on_sc=False),
    )
    def _spmm_sc(h_hbm, src_hbm, dst_hbm, zeros_hbm, out_hbm,
                 acc, srcv, dstv, rows, sem):
        c = lax.axis_index("c")
        s = lax.axis_index("s")
        wid = c * NS + s

        row0 = pl.multiple_of(s * RPT, 8)
        pltpu.sync_copy(zeros_hbm.at[pl.ds(0, RPT)], acc.at[pl.ds(row0, RPT)])

        @pl.when(s == 0)
        def _zero_tail():
            pltpu.sync_copy(zeros_hbm.at[pl.ds(0, TAIL)],
                            acc.at[pl.ds(RPT * NS, TAIL)])
        pltpu.sync_copy(src_hbm.at[wid], srcv)
        pltpu.sync_copy(dst_hbm.at[wid], dstv)
        plsc.subcore_barrier()

        def body(j, carry):
            pltpu.async_copy(h_hbm.at[srcv.at[j]], rows, sem).wait()
            pltpu.sync_copy(rows, acc.at[dstv.at[j]], add=True)
            return carry

        lax.fori_loop(0, NCHUNK, body, 0)

        plsc.subcore_barrier()
        pltpu.sync_copy(acc.at[pl.ds(row0, RPT)],
                        out_hbm.at[c].at[pl.ds(row0, RPT)])

        @pl.when(s == 0)
        def _dump_tail():
            pltpu.sync_copy(acc.at[pl.ds(RPT * NS, TAIL)],
                            out_hbm.at[c].at[pl.ds(RPT * NS, TAIL)])

    return _spmm_sc


def _emb_body(x_ref, w_ref, o_ref):
    o_ref[...] = jnp.dot(x_ref[...], w_ref[...],
                         preferred_element_type=jnp.float32)


def _layer0_body(p_ref, w_ref, b_ref, o_ref):
    agg = p_ref[0] + p_ref[1]
    o_ref[...] = jnp.maximum(
        jnp.dot(agg, w_ref[...], preferred_element_type=jnp.float32)
        + b_ref[...], 0.0)


def _layer_res_body(p_ref, w_ref, b_ref, h_ref, o_ref):
    agg = p_ref[0] + p_ref[1]
    o_ref[...] = jnp.maximum(
        jnp.dot(agg, w_ref[...], preferred_element_type=jnp.float32)
        + b_ref[...], 0.0) + h_ref[...]


def _layer_res_readout_body(p_ref, w_ref, b_ref, h_ref, o_ref, r_ref):
    agg = p_ref[0] + p_ref[1]
    out = jnp.maximum(
        jnp.dot(agg, w_ref[...], preferred_element_type=jnp.float32)
        + b_ref[...], 0.0) + h_ref[...]
    o_ref[...] = out
    r_ref[...] = jnp.sum(out, axis=0, keepdims=True)


def kernel(x, edge_index, W_emb, W0, b0, W1, b1, W2, b2):
    src = edge_index[0].reshape(NW, NCHUNK, CHUNK)
    dst = edge_index[1].reshape(NW, NCHUNK, CHUNK)
    zeros = jnp.zeros((RPT, EMB), jnp.float32)
    _spmm_sc = _make_spmm_sc()

    h = pl.pallas_call(
        _emb_body,
        out_shape=jax.ShapeDtypeStruct((N, EMB), jnp.float32),
    )(x, W_emb)

    f32 = jnp.float32
    b0 = b0.reshape(1, EMB)
    b1 = b1.reshape(1, EMB)
    b2 = b2.reshape(1, EMB)

    parts = _spmm_sc(h, src, dst, zeros)
    h = pl.pallas_call(
        _layer0_body,
        out_shape=jax.ShapeDtypeStruct((N, EMB), f32),
    )(parts, W0, b0)

    parts = _spmm_sc(h, src, dst, zeros)
    h = pl.pallas_call(
        _layer_res_body,
        out_shape=jax.ShapeDtypeStruct((N, EMB), f32),
    )(parts, W1, b1, h)

    parts = _spmm_sc(h, src, dst, zeros)
    h, readout = pl.pallas_call(
        _layer_res_readout_body,
        out_shape=(jax.ShapeDtypeStruct((N, EMB), f32),
                   jax.ShapeDtypeStruct((1, EMB), f32)),
    )(parts, W2, b2, h)

    return h, readout.reshape(EMB)


# speedup vs baseline: 12.9973x; 1.7199x over previous
import functools

import jax
import jax.numpy as jnp
from jax import lax
from jax.experimental import pallas as pl
from jax.experimental.pallas import tpu as pltpu
from jax.experimental.pallas import tpu_sc as plsc

N = 10000
E = 320000
INP = 128
EMB = 64

NC = 2
NS = 16
NW = NC * NS
EPW = E // NW
CHUNK = 125
NCHUNK = EPW // CHUNK
RPT = 624
TAIL = N - RPT * NS

@functools.cache
def _make_spmm_sc():
    mesh = plsc.VectorSubcoreMesh(
        core_axis_name="c", subcore_axis_name="s",
        num_cores=NC, num_subcores=NS)

    @functools.partial(
        pl.kernel,
        out_type=jax.ShapeDtypeStruct((NC, N, EMB), jnp.float32),
        mesh=mesh,
        scratch_types=[
            pltpu.MemorySpace.VMEM_SHARED((N, EMB), jnp.float32),
            pltpu.VMEM((NCHUNK, CHUNK), jnp.int32),
            pltpu.VMEM((NCHUNK, CHUNK), jnp.int32),
            pltpu.VMEM((CHUNK, EMB), jnp.float32),
            pltpu.VMEM((CHUNK, EMB), jnp.float32),
            pltpu.SemaphoreType.DMA,
            pltpu.SemaphoreType.DMA,
        ],
        compiler_params=pltpu.CompilerParams(use_tc_tiling_on_sc=False),
    )
    def _spmm_sc(h_hbm, src_hbm, dst_hbm, zeros_hbm, out_hbm,
                 acc, srcv, dstv, rows_a, rows_b, sem_a, sem_b):
        c = lax.axis_index("c")
        s = lax.axis_index("s")
        wid = c * NS + s

        row0 = pl.multiple_of(s * RPT, 8)
        pltpu.sync_copy(zeros_hbm.at[pl.ds(0, RPT)], acc.at[pl.ds(row0, RPT)])

        @pl.when(s == 0)
        def _zero_tail():
            pltpu.sync_copy(zeros_hbm.at[pl.ds(0, TAIL)],
                            acc.at[pl.ds(RPT * NS, TAIL)])
        pltpu.sync_copy(src_hbm.at[wid], srcv)
        pltpu.sync_copy(dst_hbm.at[wid], dstv)
        plsc.subcore_barrier()

        pltpu.async_copy(h_hbm.at[srcv.at[0]], rows_a, sem_a)

        def body(j2, carry):
            j = j2 * 2
            pltpu.async_copy(h_hbm.at[srcv.at[j + 1]], rows_b, sem_b)
            pltpu.make_async_copy(h_hbm.at[srcv.at[j]], rows_a, sem_a).wait()
            pltpu.sync_copy(rows_a, acc.at[dstv.at[j]], add=True)

            @pl.when(j + 2 < NCHUNK)
            def _next_a():
                pltpu.async_copy(h_hbm.at[srcv.at[j + 2]], rows_a, sem_a)

            pltpu.make_async_copy(
                h_hbm.at[srcv.at[j + 1]], rows_b, sem_b).wait()
            pltpu.sync_copy(rows_b, acc.at[dstv.at[j + 1]], add=True)
            return carry

        lax.fori_loop(0, NCHUNK // 2, body, 0)

        plsc.subcore_barrier()
        pltpu.sync_copy(acc.at[pl.ds(row0, RPT)],
                        out_hbm.at[c].at[pl.ds(row0, RPT)])

        @pl.when(s == 0)
        def _dump_tail():
            pltpu.sync_copy(acc.at[pl.ds(RPT * NS, TAIL)],
                            out_hbm.at[c].at[pl.ds(RPT * NS, TAIL)])

    return _spmm_sc


def _emb_body(x_ref, w_ref, o_ref):
    o_ref[...] = jnp.dot(x_ref[...], w_ref[...],
                         preferred_element_type=jnp.float32)


def _layer0_body(p_ref, w_ref, b_ref, o_ref):
    agg = p_ref[0] + p_ref[1]
    o_ref[...] = jnp.maximum(
        jnp.dot(agg, w_ref[...], preferred_element_type=jnp.float32)
        + b_ref[...], 0.0)


def _layer_res_body(p_ref, w_ref, b_ref, h_ref, o_ref):
    agg = p_ref[0] + p_ref[1]
    o_ref[...] = jnp.maximum(
        jnp.dot(agg, w_ref[...], preferred_element_type=jnp.float32)
        + b_ref[...], 0.0) + h_ref[...]


def _layer_res_readout_body(p_ref, w_ref, b_ref, h_ref, o_ref, r_ref):
    agg = p_ref[0] + p_ref[1]
    out = jnp.maximum(
        jnp.dot(agg, w_ref[...], preferred_element_type=jnp.float32)
        + b_ref[...], 0.0) + h_ref[...]
    o_ref[...] = out
    r_ref[...] = jnp.sum(out, axis=0, keepdims=True)


def kernel(x, edge_index, W_emb, W0, b0, W1, b1, W2, b2):
    src = edge_index[0].reshape(NW, NCHUNK, CHUNK)
    dst = edge_index[1].reshape(NW, NCHUNK, CHUNK)
    zeros = jnp.zeros((RPT, EMB), jnp.float32)
    _spmm_sc = _make_spmm_sc()

    h = pl.pallas_call(
        _emb_body,
        out_shape=jax.ShapeDtypeStruct((N, EMB), jnp.float32),
    )(x, W_emb)

    f32 = jnp.float32
    b0 = b0.reshape(1, EMB)
    b1 = b1.reshape(1, EMB)
    b2 = b2.reshape(1, EMB)

    parts = _spmm_sc(h, src, dst, zeros)
    h = pl.pallas_call(
        _layer0_body,
        out_shape=jax.ShapeDtypeStruct((N, EMB), f32),
    )(parts, W0, b0)

    parts = _spmm_sc(h, src, dst, zeros)
    h = pl.pallas_call(
        _layer_res_body,
        out_shape=jax.ShapeDtypeStruct((N, EMB), f32),
    )(parts, W1, b1, h)

    parts = _spmm_sc(h, src, dst, zeros)
    h, readout = pl.pallas_call(
        _layer_res_readout_body,
        out_shape=(jax.ShapeDtypeStruct((N, EMB), f32),
                   jax.ShapeDtypeStruct((1, EMB), f32)),
    )(parts, W2, b2, h)

    return h, readout.reshape(EMB)


# speedup vs baseline: 14.1112x; 1.0857x over previous
import functools

import jax
import jax.numpy as jnp
from jax import lax
from jax.experimental import pallas as pl
from jax.experimental.pallas import tpu as pltpu
from jax.experimental.pallas import tpu_sc as plsc

N = 10000
E = 320000
INP = 128
EMB = 64

NC = 2
NS = 16
NW = NC * NS
EPW = E // NW
CHUNK = 125
NCHUNK = EPW // CHUNK
RPT = 624
TAIL = N - RPT * NS

@functools.cache
def _make_spmm_sc():
    mesh = plsc.VectorSubcoreMesh(
        core_axis_name="c", subcore_axis_name="s",
        num_cores=NC, num_subcores=NS)

    @functools.partial(
        pl.kernel,
        out_type=jax.ShapeDtypeStruct((NC, N, EMB), jnp.float32),
        mesh=mesh,
        scratch_types=[
            pltpu.MemorySpace.VMEM_SHARED((N, EMB), jnp.float32),
            pltpu.VMEM((NCHUNK, CHUNK), jnp.int32),
            pltpu.VMEM((NCHUNK, CHUNK), jnp.int32),
            [pltpu.VMEM((CHUNK, EMB), jnp.float32)] * 4,
            [pltpu.SemaphoreType.DMA] * 4,
            [pltpu.SemaphoreType.DMA] * 4,
        ],
        compiler_params=pltpu.CompilerParams(use_tc_tiling_on_sc=False),
    )
    def _spmm_sc(h_hbm, src_hbm, dst_hbm, zeros_hbm, out_hbm,
                 acc, srcv, dstv, rows, gsem, ssem):
        c = lax.axis_index("c")
        s = lax.axis_index("s")
        wid = c * NS + s

        row0 = pl.multiple_of(s * RPT, 8)
        pltpu.sync_copy(zeros_hbm.at[pl.ds(0, RPT)], acc.at[pl.ds(row0, RPT)])

        @pl.when(s == 0)
        def _zero_tail():
            pltpu.sync_copy(zeros_hbm.at[pl.ds(0, TAIL)],
                            acc.at[pl.ds(RPT * NS, TAIL)])
        pltpu.sync_copy(src_hbm.at[wid], srcv)
        pltpu.sync_copy(dst_hbm.at[wid], dstv)
        plsc.subcore_barrier()

        def start_gather(b, j):
            pltpu.async_copy(h_hbm.at[srcv.at[j]], rows[b], gsem[b])

        def wait_gather(b, j):
            pltpu.make_async_copy(
                h_hbm.at[srcv.at[j]], rows[b], gsem[b]).wait()

        def start_scatter(b, j):
            pltpu.async_copy(rows[b], acc.at[dstv.at[j]], ssem[b], add=True)

        def wait_scatter(b, j):
            pltpu.make_async_copy(
                rows[b], acc.at[dstv.at[j]], ssem[b]).wait()

        start_gather(0, 0)
        start_gather(1, 1)

        def body(j4, carry):
            for i in range(4):
                j = j4 * 4 + i
                bn = (i + 2) % 4

                @pl.when(jnp.logical_and(j >= 2, j + 2 < NCHUNK))
                def _drain(bn=bn, j=j):
                    wait_scatter(bn, j - 2)

                @pl.when(j + 2 < NCHUNK)
                def _prefetch(bn=bn, j=j):
                    start_gather(bn, j + 2)

                wait_gather(i, j)
                start_scatter(i, j)
            return carry

        lax.fori_loop(0, NCHUNK // 4, body, 0)
        wait_scatter(2, NCHUNK - 2)
        wait_scatter(3, NCHUNK - 1)

        plsc.subcore_barrier()
        pltpu.sync_copy(acc.at[pl.ds(row0, RPT)],
                        out_hbm.at[c].at[pl.ds(row0, RPT)])

        @pl.when(s == 0)
        def _dump_tail():
            pltpu.sync_copy(acc.at[pl.ds(RPT * NS, TAIL)],
                            out_hbm.at[c].at[pl.ds(RPT * NS, TAIL)])

    return _spmm_sc


def _emb_body(x_ref, w_ref, o_ref):
    o_ref[...] = jnp.dot(x_ref[...], w_ref[...],
                         preferred_element_type=jnp.float32)


def _layer0_body(p_ref, w_ref, b_ref, o_ref):
    agg = p_ref[0] + p_ref[1]
    o_ref[...] = jnp.maximum(
        jnp.dot(agg, w_ref[...], preferred_element_type=jnp.float32)
        + b_ref[...], 0.0)


def _layer_res_body(p_ref, w_ref, b_ref, h_ref, o_ref):
    agg = p_ref[0] + p_ref[1]
    o_ref[...] = jnp.maximum(
        jnp.dot(agg, w_ref[...], preferred_element_type=jnp.float32)
        + b_ref[...], 0.0) + h_ref[...]


def _layer_res_readout_body(p_ref, w_ref, b_ref, h_ref, o_ref, r_ref):
    agg = p_ref[0] + p_ref[1]
    out = jnp.maximum(
        jnp.dot(agg, w_ref[...], preferred_element_type=jnp.float32)
        + b_ref[...], 0.0) + h_ref[...]
    o_ref[...] = out
    r_ref[...] = jnp.sum(out, axis=0, keepdims=True)


def kernel(x, edge_index, W_emb, W0, b0, W1, b1, W2, b2):
    src = edge_index[0].reshape(NW, NCHUNK, CHUNK)
    dst = edge_index[1].reshape(NW, NCHUNK, CHUNK)
    zeros = jnp.zeros((RPT, EMB), jnp.float32)
    _spmm_sc = _make_spmm_sc()

    h = pl.pallas_call(
        _emb_body,
        out_shape=jax.ShapeDtypeStruct((N, EMB), jnp.float32),
    )(x, W_emb)

    f32 = jnp.float32
    b0 = b0.reshape(1, EMB)
    b1 = b1.reshape(1, EMB)
    b2 = b2.reshape(1, EMB)

    parts = _spmm_sc(h, src, dst, zeros)
    h = pl.pallas_call(
        _layer0_body,
        out_shape=jax.ShapeDtypeStruct((N, EMB), f32),
    )(parts, W0, b0)

    parts = _spmm_sc(h, src, dst, zeros)
    h = pl.pallas_call(
        _layer_res_body,
        out_shape=jax.ShapeDtypeStruct((N, EMB), f32),
    )(parts, W1, b1, h)

    parts = _spmm_sc(h, src, dst, zeros)
    h, readout = pl.pallas_call(
        _layer_res_readout_body,
        out_shape=(jax.ShapeDtypeStruct((N, EMB), f32),
                   jax.ShapeDtypeStruct((1, EMB), f32)),
    )(parts, W2, b2, h)

    return h, readout.reshape(EMB)
